# Initial kernel scaffold; baseline (speedup 1.0000x reference)
import jax
import jax.numpy as jnp
from jax.experimental import pallas as pl


def kernel(inf_enc_seq, inf_enc_key_seq, e_l, e_r, start_ind, end_ind, timestep, W, b):
    raise NotImplementedError("write your pallas kernel here")


# speedup vs baseline: 224.9192x; 224.9192x over previous
import functools

import jax
import jax.numpy as jnp
from jax import lax
from jax.experimental import pallas as pl
from jax.experimental.pallas import tpu as pltpu
from jax.experimental.pallas import tpu_sc as plsc


def _make_gather(BT, D, B, T):
    info = plsc.get_sparse_core_info()
    NC, NS, L = info.num_cores, info.num_subcores, info.num_lanes
    NW = NC * NS
    assert B % NW == 0 and (B // NW) % L == 0
    b_per_w = B // NW
    mesh = plsc.VectorSubcoreMesh(core_axis_name="c", subcore_axis_name="s")

    @functools.partial(
        pl.kernel,
        mesh=mesh,
        out_type=jax.ShapeDtypeStruct((B, D), jnp.float32),
        scratch_types=[
            pltpu.VMEM((b_per_w,), jnp.int32),
            pltpu.VMEM((b_per_w, D), jnp.float32),
            pltpu.SemaphoreType.DMA,
        ],
    )
    def gather(table_hbm, ts_hbm, out_hbm, idx_v, rows_v, sem):
        wid = lax.axis_index("s") * NC + lax.axis_index("c")
        base = wid * b_per_w
        pltpu.sync_copy(ts_hbm.at[pl.ds(base, b_per_w)], idx_v)
        for g in range(b_per_w // L):
            ts = idx_v[pl.ds(g * L, L)]
            rows = base + g * L + lax.iota(jnp.int32, L)
            idx_v[pl.ds(g * L, L)] = rows * T + ts
        pltpu.async_copy(table_hbm.at[idx_v], rows_v, sem).wait()
        pltpu.sync_copy(rows_v, out_hbm.at[pl.ds(base, b_per_w)])

    return gather


def _mm_kernel(el_ref, er_ref, et_ref, w_ref, b_ref, mu_ref, ls_ref):
    D = el_ref.shape[1]
    NZ = mu_ref.shape[1]
    h = (
        jnp.dot(el_ref[...], w_ref[0:D, :], preferred_element_type=jnp.float32)
        + jnp.dot(er_ref[...], w_ref[D : 2 * D, :], preferred_element_type=jnp.float32)
        + jnp.dot(et_ref[...], w_ref[2 * D : 3 * D, :], preferred_element_type=jnp.float32)
        + b_ref[...]
    )
    mu_ref[...] = h[:, :NZ]
    ls_ref[...] = h[:, NZ:]


def kernel(inf_enc_seq, inf_enc_key_seq, e_l, e_r, start_ind, end_ind, timestep, W, b):
    B, T, D = inf_enc_seq.shape
    NZ = W.shape[1] // 2
    table = inf_enc_seq.reshape(B * T, D)
    ts = timestep.reshape(B).astype(jnp.int32)
    e_tilde = _make_gather(B * T, D, B, T)(table, ts)
    mu, log_sigma = pl.pallas_call(
        _mm_kernel,
        out_shape=(
            jax.ShapeDtypeStruct((B, NZ), jnp.float32),
            jax.ShapeDtypeStruct((B, NZ), jnp.float32),
        ),
    )(e_l, e_r, e_tilde, W, b.reshape(1, 2 * NZ))
    return (mu, log_sigma)


# speedup vs baseline: 526.5262x; 2.3410x over previous
import jax
import jax.numpy as jnp
from jax import lax
from jax.experimental import pallas as pl
from jax.experimental.pallas import tpu as pltpu


def _fused_kernel(ts_ref, seq_ref, el_ref, er_ref, w_ref, b_ref,
                  mu_ref, ls_ref, et_ref, sem):
    B, D = et_ref.shape
    NZ = mu_ref.shape[1]

    def issue(i, carry):
        t = ts_ref[i]
        pltpu.make_async_copy(seq_ref.at[i, t], et_ref.at[i], sem).start()
        return carry

    lax.fori_loop(0, B, issue, 0, unroll=8)

    part = (
        jnp.dot(el_ref[...], w_ref[0:D, :], preferred_element_type=jnp.float32)
        + jnp.dot(er_ref[...], w_ref[D : 2 * D, :], preferred_element_type=jnp.float32)
        + b_ref[...]
    )

    def drain(i, carry):
        t = ts_ref[i]
        pltpu.make_async_copy(seq_ref.at[i, t], et_ref.at[i], sem).wait()
        return carry

    lax.fori_loop(0, B, drain, 0, unroll=8)

    h = part + jnp.dot(
        et_ref[...], w_ref[2 * D : 3 * D, :], preferred_element_type=jnp.float32
    )
    mu_ref[...] = h[:, :NZ]
    ls_ref[...] = h[:, NZ:]


def kernel(inf_enc_seq, inf_enc_key_seq, e_l, e_r, start_ind, end_ind, timestep, W, b):
    B, T, D = inf_enc_seq.shape
    NZ = W.shape[1] // 2
    ts = timestep.reshape(B).astype(jnp.int32)
    mu, log_sigma = pl.pallas_call(
        _fused_kernel,
        in_specs=[
            pl.BlockSpec(memory_space=pltpu.SMEM),
            pl.BlockSpec(memory_space=pl.ANY),
            pl.BlockSpec(memory_space=pltpu.VMEM),
            pl.BlockSpec(memory_space=pltpu.VMEM),
            pl.BlockSpec(memory_space=pltpu.VMEM),
            pl.BlockSpec(memory_space=pltpu.VMEM),
        ],
        out_specs=(
            pl.BlockSpec(memory_space=pltpu.VMEM),
            pl.BlockSpec(memory_space=pltpu.VMEM),
        ),
        out_shape=(
            jax.ShapeDtypeStruct((B, NZ), jnp.float32),
            jax.ShapeDtypeStruct((B, NZ), jnp.float32),
        ),
        scratch_shapes=[
            pltpu.VMEM((B, D), jnp.float32),
            pltpu.SemaphoreType.DMA,
        ],
    )(ts, inf_enc_seq, e_l, e_r, W, b.reshape(1, 2 * NZ))
    return (mu, log_sigma)
